# Initial kernel scaffold; baseline (speedup 1.0000x reference)
import jax
import jax.numpy as jnp
from jax.experimental import pallas as pl


def kernel(x, target):
    raise NotImplementedError("write your pallas kernel here")


# speedup vs baseline: 1.0182x; 1.0182x over previous
import functools

import jax
import jax.numpy as jnp
from jax import lax
from jax.experimental import pallas as pl
from jax.experimental.pallas import tpu as pltpu
from jax.experimental.pallas import tpu_sc as plsc

B = 1024
V = 100000
L = 16
NC = 2
NS = 16
NW = NC * NS
CHUNK = B // NW


def _kl_body(x_hbm, tgt_hbm, out_hbm, tgt_v, idx_v, val_v, acc_v, all_v,
             shared, sem):
    cid = lax.axis_index("c")
    sid = lax.axis_index("s")
    wid = cid * NS + sid
    base = wid * CHUNK

    pltpu.sync_copy(tgt_hbm.at[pl.ds(base, CHUNK)], tgt_v)

    lane = lax.iota(jnp.int32, L)
    for h in range(CHUNK // L):
        t = tgt_v[pl.ds(h * L, L)]
        rows = (base + h * L) + lane
        idx_v[pl.ds(h * L, L)] = rows * V + t

    pltpu.async_copy(x_hbm.at[idx_v], val_v, sem).wait()

    acc = jnp.zeros((L,), jnp.float32)
    for h in range(CHUNK // L):
        t = tgt_v[pl.ds(h * L, L)]
        v = val_v[pl.ds(h * L, L)]
        acc = acc + jnp.where(t != 0, -v, 0.0)
    acc_v[...] = acc

    pltpu.sync_copy(acc_v, shared.at[pl.ds(sid * L, L)])
    plsc.subcore_barrier()

    @pl.when(sid == 0)
    def _():
        pltpu.sync_copy(shared, all_v)
        s = all_v[pl.ds(0, L)]
        for j in range(1, NS):
            s = s + all_v[pl.ds(j * L, L)]
        acc_v[...] = s
        pltpu.sync_copy(acc_v, out_hbm.at[cid])


_kl = functools.partial(
    pl.kernel,
    out_type=jax.ShapeDtypeStruct((NC, L), jnp.float32),
    mesh=plsc.VectorSubcoreMesh(core_axis_name="c", subcore_axis_name="s"),
    scratch_types=[
        pltpu.VMEM((CHUNK,), jnp.int32),
        pltpu.VMEM((CHUNK,), jnp.int32),
        pltpu.VMEM((CHUNK,), jnp.float32),
        pltpu.VMEM((L,), jnp.float32),
        pltpu.VMEM((NS * L,), jnp.float32),
        pltpu.VMEM_SHARED((NS * L,), jnp.float32),
        pltpu.SemaphoreType.DMA,
    ],
)(_kl_body)


def kernel(x, target):
    partials = _kl(x.reshape(-1), target)
    return jnp.sum(partials)


# speedup vs baseline: 2.4014x; 2.3585x over previous
import functools

import jax
import jax.numpy as jnp
from jax import lax
from jax.experimental import pallas as pl
from jax.experimental.pallas import tpu as pltpu
from jax.experimental.pallas import tpu_sc as plsc

B = 1024
V = 100000
L = 16
NC = 2
NS = 16
NW = NC * NS
CHUNK = B // NW


def _kl_body(x_hbm, tgt_hbm, out_hbm, tgt_v, tiles_v, acc_v, all_v, shared,
             sem):
    cid = lax.axis_index("c")
    sid = lax.axis_index("s")
    wid = cid * NS + sid
    base = wid * CHUNK

    pltpu.sync_copy(tgt_hbm.at[pl.ds(base, CHUNK)], tgt_v)
    lane = lax.iota(jnp.int32, L)

    handles = []
    for h in range(CHUNK // L):
        t = tgt_v[pl.ds(h * L, L)]
        for j in range(L):
            r = h * L + j
            tr = lax.squeeze(lax.slice(t, (j,), (j + 1,)), (0,))
            c0 = pl.multiple_of((tr >> 7) << 7, 128)
            r0 = pl.multiple_of(base + (r // 8) * 8, 8)
            handles.append(pltpu.async_copy(
                x_hbm.at[pl.ds(r0, 8), pl.ds(c0, 128)],
                tiles_v.at[r], sem))
    for hd in handles:
        hd.wait()

    acc = jnp.zeros((L,), jnp.float32)
    for h in range(CHUNK // L):
        t = tgt_v[pl.ds(h * L, L)]
        buf = h * L + lane
        sub = lane & 7
        col = t & 127
        v = plsc.load_gather(tiles_v, [buf, sub, col])
        acc = acc + jnp.where(t != 0, -v, 0.0)
    acc_v[...] = acc

    pltpu.sync_copy(acc_v, shared.at[pl.ds(sid * L, L)])
    plsc.subcore_barrier()

    @pl.when(sid == 0)
    def _():
        pltpu.sync_copy(shared, all_v)
        s = all_v[pl.ds(0, L)]
        for j in range(1, NS):
            s = s + all_v[pl.ds(j * L, L)]
        acc_v[...] = s
        pltpu.sync_copy(acc_v, out_hbm.at[cid])


_kl = functools.partial(
    pl.kernel,
    out_type=jax.ShapeDtypeStruct((NC, L), jnp.float32),
    mesh=plsc.VectorSubcoreMesh(core_axis_name="c", subcore_axis_name="s"),
    compiler_params=pltpu.CompilerParams(needs_layout_passes=False),
    scratch_types=[
        pltpu.VMEM((CHUNK,), jnp.int32),
        pltpu.VMEM((CHUNK, 8, 128), jnp.float32),
        pltpu.VMEM((L,), jnp.float32),
        pltpu.VMEM((NS * L,), jnp.float32),
        pltpu.VMEM_SHARED((NS * L,), jnp.float32),
        pltpu.SemaphoreType.DMA,
    ],
)(_kl_body)


def kernel(x, target):
    partials = _kl(x, target)
    return jnp.sum(partials)


# speedup vs baseline: 35.9352x; 14.9640x over previous
import functools

import jax
import jax.numpy as jnp
from jax import lax
from jax.experimental import pallas as pl
from jax.experimental.pallas import tpu as pltpu
from jax.experimental.pallas import tpu_sc as plsc

B = 1024
V = 100000
L = 16
NC = 2
NS = 16
NW = NC * NS
CHUNK = B // NW


def _kl_body(xt_hbm, tgt_hbm, out_hbm, tgt_v, tiles_v, acc_v, all_v, shared,
             sem):
    cid = lax.axis_index("c")
    sid = lax.axis_index("s")
    wid = cid * NS + sid
    base = wid * CHUNK

    pltpu.sync_copy(tgt_hbm.at[pl.ds(base, CHUNK)], tgt_v)
    lane = lax.iota(jnp.int32, L)

    c0 = pl.multiple_of((base // 128) * 128, 128)

    handles = []
    for h in range(CHUNK // L):
        t = tgt_v[pl.ds(h * L, L)]
        for j in range(L):
            r = h * L + j
            tr = lax.squeeze(lax.slice(t, (j,), (j + 1,)), (0,))
            t0 = pl.multiple_of((tr >> 3) << 3, 8)
            handles.append(pltpu.async_copy(
                xt_hbm.at[pl.ds(t0, 8), pl.ds(c0, 128)],
                tiles_v.at[r], sem))
    for hd in handles:
        hd.wait()

    colbase = base - (base // 128) * 128
    acc = jnp.zeros((L,), jnp.float32)
    for h in range(CHUNK // L):
        t = tgt_v[pl.ds(h * L, L)]
        buf = h * L + lane
        sub = t & 7
        col = colbase + h * L + lane
        v = plsc.load_gather(tiles_v, [buf, sub, col])
        acc = acc + jnp.where(t != 0, -v, 0.0)
    acc_v[...] = acc

    pltpu.sync_copy(acc_v, shared.at[pl.ds(sid * L, L)])
    plsc.subcore_barrier()

    @pl.when(sid == 0)
    def _():
        pltpu.sync_copy(shared, all_v)
        s = all_v[pl.ds(0, L)]
        for j in range(1, NS):
            s = s + all_v[pl.ds(j * L, L)]
        acc_v[...] = s
        pltpu.sync_copy(acc_v, out_hbm.at[cid])


_kl = functools.partial(
    pl.kernel,
    out_type=jax.ShapeDtypeStruct((NC, L), jnp.float32),
    mesh=plsc.VectorSubcoreMesh(core_axis_name="c", subcore_axis_name="s"),
    compiler_params=pltpu.CompilerParams(needs_layout_passes=False),
    scratch_types=[
        pltpu.VMEM((CHUNK,), jnp.int32),
        pltpu.VMEM((CHUNK, 8, 128), jnp.float32),
        pltpu.VMEM((L,), jnp.float32),
        pltpu.VMEM((NS * L,), jnp.float32),
        pltpu.VMEM_SHARED((NS * L,), jnp.float32),
        pltpu.SemaphoreType.DMA,
    ],
)(_kl_body)


def kernel(x, target):
    partials = _kl(x.T, target)
    return jnp.sum(partials)
